# Initial kernel scaffold; baseline (speedup 1.0000x reference)
import jax
import jax.numpy as jnp
from jax.experimental import pallas as pl


def kernel(means, covariance_matrices, colors, opacities, R, t, fx, fy, cx, cy, width, height):
    raise NotImplementedError("write your pallas kernel here")


# speedup vs baseline: 130.9318x; 130.9318x over previous
import functools

import jax
import jax.numpy as jnp
from jax.experimental import pallas as pl
from jax.experimental.pallas import tpu as pltpu

W = 512
H = 512
BH = 64
NBANDS = H // BH
NP = 2048

P_ROWS = 16


def _composite_kernel(n_real, params_ref, out_ref):
    band = pl.program_id(0)
    y0f = (band * BH).astype(jnp.float32)
    y1f = y0f + float(BH)
    ys = jax.lax.broadcasted_iota(jnp.int32, (BH, W), 0).astype(jnp.float32) + y0f
    xs = jax.lax.broadcasted_iota(jnp.int32, (BH, W), 1).astype(jnp.float32)
    out_ref[...] = jnp.zeros((3, BH, W), jnp.float32)

    def body(j, carry):
        xmin = params_ref[9, j]
        xmax = params_ref[10, j]
        ymin = params_ref[11, j]
        ymax = params_ref[12, j]
        op = params_ref[5, j]
        pred = (ymax > y0f) & (ymin < y1f) & (xmax > xmin) & (op > 0.0)

        @pl.when(pred)
        def _():
            u = params_ref[0, j]
            v = params_ref[1, j]
            pa = params_ref[2, j]
            pb = params_ref[3, j]
            pc = params_ref[4, j]
            cr = params_ref[6, j]
            cg = params_ref[7, j]
            cb = params_ref[8, j]
            dx = xs - u
            dy = ys - v
            e = (pa * dx) * dx + (pb * dx) * dy + (pc * dy) * dy
            alpha = jnp.exp(e) * op
            mask = (xs >= xmin) & (xs < xmax) & (ys >= ymin) & (ys < ymax)
            alpha = jnp.where(mask, alpha, 0.0)
            om = 1.0 - alpha
            out_ref[0, :, :] = cr * alpha + out_ref[0, :, :] * om
            out_ref[1, :, :] = cg * alpha + out_ref[1, :, :] * om
            out_ref[2, :, :] = cb * alpha + out_ref[2, :, :] * om

        return carry

    jax.lax.fori_loop(0, n_real, body, 0)


def kernel(means, covariance_matrices, colors, opacities, R, t, fx, fy, cx, cy, width, height):
    n = means.shape[0]
    means_cam = jnp.matmul(R, means.T).T + t
    z = means_cam[:, 2]
    u = means_cam[:, 0] / z * fx + cx
    v = means_cam[:, 1] / z * fy + cy
    cov_cam = jnp.einsum('ij,njk,lk->nil', R, covariance_matrices, R)
    zeros = jnp.zeros_like(z)
    J = jnp.stack([
        jnp.stack([fx / z, zeros, -fx * means_cam[:, 0] / (z * z)], axis=1),
        jnp.stack([zeros, fy / z, -fy * means_cam[:, 1] / (z * z)], axis=1),
    ], axis=1)
    cov_2d = jnp.einsum('nij,njk,nlk->nil', J, cov_cam, J)

    order = jnp.argsort(-z, stable=True)
    a = cov_2d[:, 0, 0][order]
    b = cov_2d[:, 0, 1][order]
    c = cov_2d[:, 1, 1][order]
    u = u[order]
    v = v[order]
    zs = z[order]
    cols = colors[order]
    ops = opacities[order]

    det = a * c - b * b
    pa = -0.5 * c / det
    pb = b / det
    pc = -0.5 * a / det

    m = 0.5 * (a + c)
    d = 0.5 * (a - c)
    r = jnp.sqrt(d * d + b * b)
    s0 = 3.0 * jnp.sqrt(jnp.abs(m - r))
    s1 = 3.0 * jnp.sqrt(jnp.abs(m + r))
    rd = r - d
    vn = jnp.sqrt(b * b + rd * rd)
    safe = vn > 1e-20
    iv = 1.0 / jnp.where(safe, vn, 1.0)
    ext_x = jnp.where(safe, jnp.maximum(s0 * rd, s1 * jnp.abs(b)) * iv, s1)
    ext_y = jnp.where(safe, jnp.maximum(s0 * jnp.abs(b), s1 * rd) * iv, s0)

    width_f = jnp.asarray(width, jnp.float32)
    height_f = jnp.asarray(height, jnp.float32)
    xmin = jnp.maximum(0.0, jnp.floor(u - ext_x))
    xmax = jnp.minimum(width_f, jnp.ceil(u + ext_x))
    ymin = jnp.maximum(0.0, jnp.floor(v - ext_y))
    ymax = jnp.minimum(height_f, jnp.ceil(v + ext_y))

    op_live = ops * (zs > 0.0).astype(jnp.float32)

    params = jnp.stack([
        u, v, pa, pb, pc, op_live, cols[:, 0], cols[:, 1], cols[:, 2],
        xmin, xmax, ymin, ymax,
        jnp.zeros_like(u), jnp.zeros_like(u), jnp.zeros_like(u),
    ])
    params = jnp.pad(params, ((0, 0), (0, NP - n)))

    img = pl.pallas_call(
        functools.partial(_composite_kernel, n),
        grid=(NBANDS,),
        in_specs=[pl.BlockSpec((P_ROWS, NP), lambda i: (0, 0),
                               memory_space=pltpu.SMEM)],
        out_specs=pl.BlockSpec((3, BH, W), lambda i: (0, i, 0)),
        out_shape=jax.ShapeDtypeStruct((3, H, W), jnp.float32),
    )(params)
    return jnp.transpose(img, (1, 2, 0))


# speedup vs baseline: 340.2995x; 2.5991x over previous
import functools

import jax
import jax.numpy as jnp
from jax import lax
from jax.experimental import pallas as pl
from jax.experimental.pallas import tpu as pltpu
from jax.experimental.pallas import tpu_sc as plsc

W = 512
H = 512
BW = 128
BH = 64
TX = W // BW
TY = H // BH
NT = TX * TY
NP = 2048
L = 16

P_ROWS = 16
BB_ROWS = 8


def _binning_body(bb_hbm, idx_hbm, cnt_hbm, bb_v, idx_v, cnt_v):
    nc = 2
    wid = lax.axis_index("s") * nc + lax.axis_index("c")
    ty = wid // TX
    tx = wid - ty * TX
    x0 = (tx * BW).astype(jnp.float32)
    x1 = x0 + float(BW)
    y0 = (ty * BH).astype(jnp.float32)
    y1 = y0 + float(BH)

    pltpu.sync_copy(bb_hbm, bb_v)

    ones = jnp.full((L,), 1, jnp.int32)
    zeros_i = jnp.full((L,), 0, jnp.int32)
    trash = jnp.full((L,), NP + L, jnp.int32)
    x1v = jnp.full((L,), x1, jnp.float32)
    x0v = jnp.full((L,), x0, jnp.float32)
    y1v = jnp.full((L,), y1, jnp.float32)
    y0v = jnp.full((L,), y0, jnp.float32)
    zf = jnp.full((L,), 0.0, jnp.float32)
    lane = lax.broadcasted_iota(jnp.int32, (L,), 0)

    def chunk(i, cnt):
        sl = pl.ds(i * L, L)
        xmin = bb_v[0, sl]
        xmax = bb_v[1, sl]
        ymin = bb_v[2, sl]
        ymax = bb_v[3, sl]
        op = bb_v[4, sl]
        pred = ((xmin < x1v) & (xmax > x0v) & (ymin < y1v) & (ymax > y0v)
                & (op > zf))
        predi = jnp.where(pred, ones, zeros_i)
        csum = plsc.cumsum(predi)
        cntv = jnp.full((L,), cnt, jnp.int32)
        ids = lane + jnp.full((L,), i * L, jnp.int32)
        dst = jnp.where(pred, cntv + csum - predi, trash)
        plsc.store_scatter(idx_v, [dst], ids)
        return cnt + jnp.sum(predi)

    cnt = lax.fori_loop(0, NP // L, chunk, jnp.int32(0))
    cnt_v[...] = jnp.full((L,), cnt, jnp.int32)
    pltpu.sync_copy(idx_v.at[pl.ds(0, NP)], idx_hbm.at[wid])
    pltpu.sync_copy(cnt_v, cnt_hbm.at[wid, pl.ds(0, L)])


_binning_kernel = functools.partial(
    pl.kernel,
    mesh=plsc.VectorSubcoreMesh(core_axis_name="c", subcore_axis_name="s"),
    compiler_params=pltpu.CompilerParams(needs_layout_passes=False),
    out_type=[
        jax.ShapeDtypeStruct((NT, NP), jnp.int32),
        jax.ShapeDtypeStruct((NT, 128), jnp.int32),
    ],
    scratch_types=[
        pltpu.VMEM((BB_ROWS, NP), jnp.float32),
        pltpu.VMEM((NP + 2 * L,), jnp.int32),
        pltpu.VMEM((L,), jnp.int32),
    ],
)(_binning_body)


def _composite_kernel(params_ref, idx_ref, cnt_ref, out_ref):
    ty = pl.program_id(0)
    tx = pl.program_id(1)
    y0f = (ty * BH).astype(jnp.float32)
    x0f = (tx * BW).astype(jnp.float32)
    ys = lax.broadcasted_iota(jnp.int32, (BH, BW), 0).astype(jnp.float32) + y0f
    xs = lax.broadcasted_iota(jnp.int32, (BH, BW), 1).astype(jnp.float32) + x0f
    out_ref[...] = jnp.zeros((3, BH, BW), jnp.float32)
    n = cnt_ref[0]

    def body(j, carry):
        g = idx_ref[j]
        u = params_ref[0, g]
        v = params_ref[1, g]
        pa = params_ref[2, g]
        pb = params_ref[3, g]
        pc = params_ref[4, g]
        op = params_ref[5, g]
        cr = params_ref[6, g]
        cg = params_ref[7, g]
        cb = params_ref[8, g]
        xmin = params_ref[9, g]
        xmax = params_ref[10, g]
        ymin = params_ref[11, g]
        ymax = params_ref[12, g]
        dx = xs - u
        dy = ys - v
        e = (pa * dx) * dx + (pb * dx) * dy + (pc * dy) * dy
        alpha = jnp.exp(e) * op
        mask = (xs >= xmin) & (xs < xmax) & (ys >= ymin) & (ys < ymax)
        alpha = jnp.where(mask, alpha, 0.0)
        om = 1.0 - alpha
        out_ref[0, :, :] = cr * alpha + out_ref[0, :, :] * om
        out_ref[1, :, :] = cg * alpha + out_ref[1, :, :] * om
        out_ref[2, :, :] = cb * alpha + out_ref[2, :, :] * om
        return carry

    lax.fori_loop(0, n, body, 0)


def kernel(means, covariance_matrices, colors, opacities, R, t, fx, fy, cx, cy, width, height):
    n = means.shape[0]
    means_cam = jnp.matmul(R, means.T).T + t
    z = means_cam[:, 2]
    u = means_cam[:, 0] / z * fx + cx
    v = means_cam[:, 1] / z * fy + cy
    cov_cam = jnp.einsum('ij,njk,lk->nil', R, covariance_matrices, R)
    zeros = jnp.zeros_like(z)
    J = jnp.stack([
        jnp.stack([fx / z, zeros, -fx * means_cam[:, 0] / (z * z)], axis=1),
        jnp.stack([zeros, fy / z, -fy * means_cam[:, 1] / (z * z)], axis=1),
    ], axis=1)
    cov_2d = jnp.einsum('nij,njk,nlk->nil', J, cov_cam, J)

    order = jnp.argsort(-z, stable=True)
    a = cov_2d[:, 0, 0][order]
    b = cov_2d[:, 0, 1][order]
    c = cov_2d[:, 1, 1][order]
    u = u[order]
    v = v[order]
    zs = z[order]
    cols = colors[order]
    ops = opacities[order]

    det = a * c - b * b
    pa = -0.5 * c / det
    pb = b / det
    pc = -0.5 * a / det

    m = 0.5 * (a + c)
    d = 0.5 * (a - c)
    r = jnp.sqrt(d * d + b * b)
    s0 = 3.0 * jnp.sqrt(jnp.abs(m - r))
    s1 = 3.0 * jnp.sqrt(jnp.abs(m + r))
    rd = r - d
    vn = jnp.sqrt(b * b + rd * rd)
    safe = vn > 1e-20
    iv = 1.0 / jnp.where(safe, vn, 1.0)
    ext_x = jnp.where(safe, jnp.maximum(s0 * rd, s1 * jnp.abs(b)) * iv, s1)
    ext_y = jnp.where(safe, jnp.maximum(s0 * jnp.abs(b), s1 * rd) * iv, s0)

    width_f = jnp.asarray(width, jnp.float32)
    height_f = jnp.asarray(height, jnp.float32)
    xmin = jnp.maximum(0.0, jnp.floor(u - ext_x))
    xmax = jnp.minimum(width_f, jnp.ceil(u + ext_x))
    ymin = jnp.maximum(0.0, jnp.floor(v - ext_y))
    ymax = jnp.minimum(height_f, jnp.ceil(v + ext_y))

    op_live = ops * (zs > 0.0).astype(jnp.float32)

    zero = jnp.zeros_like(u)
    params = jnp.stack([
        u, v, pa, pb, pc, op_live, cols[:, 0], cols[:, 1], cols[:, 2],
        xmin, xmax, ymin, ymax, zero, zero, zero,
    ])
    params = jnp.pad(params, ((0, 0), (0, NP - n)))

    bb = jnp.stack([xmin, xmax, ymin, ymax, op_live, zero, zero, zero])
    bb = jnp.pad(bb, ((0, 0), (0, NP - n)))

    idx_lists, cnts = _binning_kernel(bb)
    idx_flat = jnp.reshape(idx_lists, (NT * NP,))
    cnt_flat = jnp.reshape(cnts, (NT * 128,))

    img = pl.pallas_call(
        _composite_kernel,
        grid=(TY, TX),
        in_specs=[
            pl.BlockSpec((P_ROWS, NP), lambda ty, tx: (0, 0),
                         memory_space=pltpu.SMEM),
            pl.BlockSpec((NP,), lambda ty, tx: (ty * TX + tx,),
                         memory_space=pltpu.SMEM),
            pl.BlockSpec((128,), lambda ty, tx: (ty * TX + tx,),
                         memory_space=pltpu.SMEM),
        ],
        out_specs=pl.BlockSpec((3, BH, BW), lambda ty, tx: (0, ty, tx)),
        out_shape=jax.ShapeDtypeStruct((3, H, W), jnp.float32),
    )(params, idx_flat, cnt_flat)
    return jnp.transpose(img, (1, 2, 0))


# speedup vs baseline: 468.9681x; 1.3781x over previous
import functools

import jax
import jax.numpy as jnp
from jax import lax
from jax.experimental import pallas as pl
from jax.experimental.pallas import tpu as pltpu
from jax.experimental.pallas import tpu_sc as plsc

W = 512
H = 512
BW = 128
BH = 64
TX = W // BW
TY = H // BH
NT = TX * TY
NP = 2048
L = 16

P_COLS = 16


def _binning_body(ptab_hbm, ord_hbm, idx_hbm, cnt_hbm, ptab_v, ord_v, idx_v, cnt_v):
    nc = 2
    wid = lax.axis_index("s") * nc + lax.axis_index("c")
    ty = wid // TX
    tx = wid - ty * TX
    x0 = (tx * BW).astype(jnp.float32)
    x1 = x0 + float(BW)
    y0 = (ty * BH).astype(jnp.float32)
    y1 = y0 + float(BH)

    pltpu.sync_copy(ptab_hbm, ptab_v)
    pltpu.sync_copy(ord_hbm, ord_v)

    ones = jnp.full((L,), 1, jnp.int32)
    zeros_i = jnp.full((L,), 0, jnp.int32)
    trash = jnp.full((L,), NP + L, jnp.int32)
    x1v = jnp.full((L,), x1, jnp.float32)
    x0v = jnp.full((L,), x0, jnp.float32)
    y1v = jnp.full((L,), y1, jnp.float32)
    y0v = jnp.full((L,), y0, jnp.float32)
    zf = jnp.full((L,), 0.0, jnp.float32)
    c_xmin = jnp.full((L,), 9 * NP, jnp.int32)
    c_xmax = jnp.full((L,), 10 * NP, jnp.int32)
    c_ymin = jnp.full((L,), 11 * NP, jnp.int32)
    c_ymax = jnp.full((L,), 12 * NP, jnp.int32)
    c_op = jnp.full((L,), 5 * NP, jnp.int32)

    def chunk(i, cnt):
        gids = ord_v[pl.ds(i * L, L)]
        xmin = plsc.load_gather(ptab_v, [gids + c_xmin])
        xmax = plsc.load_gather(ptab_v, [gids + c_xmax])
        ymin = plsc.load_gather(ptab_v, [gids + c_ymin])
        ymax = plsc.load_gather(ptab_v, [gids + c_ymax])
        op = plsc.load_gather(ptab_v, [gids + c_op])
        pred = ((xmin < x1v) & (xmax > x0v) & (ymin < y1v) & (ymax > y0v)
                & (op > zf))
        predi = jnp.where(pred, ones, zeros_i)
        csum = plsc.cumsum(predi)
        cntv = jnp.full((L,), cnt, jnp.int32)
        dst = jnp.where(pred, cntv + csum - predi, trash)
        plsc.store_scatter(idx_v, [dst], gids)
        return cnt + jnp.sum(predi)

    cnt = lax.fori_loop(0, NP // L, chunk, jnp.int32(0))
    cnt_v[...] = jnp.full((L,), cnt, jnp.int32)
    pltpu.sync_copy(idx_v.at[pl.ds(0, NP)], idx_hbm.at[wid])
    pltpu.sync_copy(cnt_v, cnt_hbm.at[wid, pl.ds(0, L)])


_binning_kernel = functools.partial(
    pl.kernel,
    mesh=plsc.VectorSubcoreMesh(core_axis_name="c", subcore_axis_name="s"),
    compiler_params=pltpu.CompilerParams(needs_layout_passes=False),
    out_type=[
        jax.ShapeDtypeStruct((NT, NP), jnp.int32),
        jax.ShapeDtypeStruct((NT, 128), jnp.int32),
    ],
    scratch_types=[
        pltpu.VMEM((P_COLS * NP,), jnp.float32),
        pltpu.VMEM((NP,), jnp.int32),
        pltpu.VMEM((NP + 2 * L,), jnp.int32),
        pltpu.VMEM((L,), jnp.int32),
    ],
)(_binning_body)


def _composite_kernel(params_ref, idx_ref, cnt_ref, out_ref):
    ty = pl.program_id(0)
    tx = pl.program_id(1)
    y0f = (ty * BH).astype(jnp.float32)
    x0f = (tx * BW).astype(jnp.float32)
    ys = lax.broadcasted_iota(jnp.int32, (BH, BW), 0).astype(jnp.float32) + y0f
    xs = lax.broadcasted_iota(jnp.int32, (BH, BW), 1).astype(jnp.float32) + x0f
    out_ref[...] = jnp.zeros((3, BH, BW), jnp.float32)
    n = cnt_ref[0]

    def body(j, carry):
        g = idx_ref[j]
        u = params_ref[0, g]
        v = params_ref[1, g]
        pa = params_ref[2, g]
        pb = params_ref[3, g]
        pc = params_ref[4, g]
        op = params_ref[5, g]
        cr = params_ref[6, g]
        cg = params_ref[7, g]
        cb = params_ref[8, g]
        xmin = params_ref[9, g]
        xmax = params_ref[10, g]
        ymin = params_ref[11, g]
        ymax = params_ref[12, g]
        dx = xs - u
        dy = ys - v
        e = (pa * dx) * dx + (pb * dx) * dy + (pc * dy) * dy
        alpha = jnp.exp(e) * op
        mask = (xs >= xmin) & (xs < xmax) & (ys >= ymin) & (ys < ymax)
        alpha = jnp.where(mask, alpha, 0.0)
        om = 1.0 - alpha
        out_ref[0, :, :] = cr * alpha + out_ref[0, :, :] * om
        out_ref[1, :, :] = cg * alpha + out_ref[1, :, :] * om
        out_ref[2, :, :] = cb * alpha + out_ref[2, :, :] * om
        return carry

    lax.fori_loop(0, n, body, 0)


def kernel(means, covariance_matrices, colors, opacities, R, t, fx, fy, cx, cy, width, height):
    n = means.shape[0]
    means_cam = jnp.matmul(R, means.T).T + t
    z = means_cam[:, 2]
    u = means_cam[:, 0] / z * fx + cx
    v = means_cam[:, 1] / z * fy + cy
    cov_cam = jnp.einsum('ij,njk,lk->nil', R, covariance_matrices, R)
    zeros = jnp.zeros_like(z)
    J = jnp.stack([
        jnp.stack([fx / z, zeros, -fx * means_cam[:, 0] / (z * z)], axis=1),
        jnp.stack([zeros, fy / z, -fy * means_cam[:, 1] / (z * z)], axis=1),
    ], axis=1)
    cov_2d = jnp.einsum('nij,njk,nlk->nil', J, cov_cam, J)

    a = cov_2d[:, 0, 0]
    b = cov_2d[:, 0, 1]
    c = cov_2d[:, 1, 1]

    det = a * c - b * b
    pa = -0.5 * c / det
    pb = b / det
    pc = -0.5 * a / det

    m = 0.5 * (a + c)
    d = 0.5 * (a - c)
    r = jnp.sqrt(d * d + b * b)
    s0 = 3.0 * jnp.sqrt(jnp.abs(m - r))
    s1 = 3.0 * jnp.sqrt(jnp.abs(m + r))
    rd = r - d
    vn = jnp.sqrt(b * b + rd * rd)
    safe = vn > 1e-20
    iv = 1.0 / jnp.where(safe, vn, 1.0)
    ext_x = jnp.where(safe, jnp.maximum(s0 * rd, s1 * jnp.abs(b)) * iv, s1)
    ext_y = jnp.where(safe, jnp.maximum(s0 * jnp.abs(b), s1 * rd) * iv, s0)

    width_f = jnp.asarray(width, jnp.float32)
    height_f = jnp.asarray(height, jnp.float32)
    xmin = jnp.maximum(0.0, jnp.floor(u - ext_x))
    xmax = jnp.minimum(width_f, jnp.ceil(u + ext_x))
    ymin = jnp.maximum(0.0, jnp.floor(v - ext_y))
    ymax = jnp.minimum(height_f, jnp.ceil(v + ext_y))

    op_live = opacities * (z > 0.0).astype(jnp.float32)

    order = jnp.argsort(-z, stable=True).astype(jnp.int32)
    order = jnp.pad(order, (0, NP - n), constant_values=NP - 1)

    zero = jnp.zeros_like(u)
    params = jnp.stack([
        u, v, pa, pb, pc, op_live, colors[:, 0], colors[:, 1], colors[:, 2],
        xmin, xmax, ymin, ymax, zero, zero, zero,
    ])
    params = jnp.pad(params, ((0, 0), (0, NP - n)))

    idx_lists, cnts = _binning_kernel(jnp.reshape(params, (P_COLS * NP,)), order)
    idx_flat = jnp.reshape(idx_lists, (NT * NP,))
    cnt_flat = jnp.reshape(cnts, (NT * 128,))

    img = pl.pallas_call(
        _composite_kernel,
        grid=(TY, TX),
        in_specs=[
            pl.BlockSpec((P_COLS, NP), lambda ty, tx: (0, 0),
                         memory_space=pltpu.SMEM),
            pl.BlockSpec((NP,), lambda ty, tx: (ty * TX + tx,),
                         memory_space=pltpu.SMEM),
            pl.BlockSpec((128,), lambda ty, tx: (ty * TX + tx,),
                         memory_space=pltpu.SMEM),
        ],
        out_specs=pl.BlockSpec((3, BH, BW), lambda ty, tx: (0, ty, tx)),
        out_shape=jax.ShapeDtypeStruct((3, H, W), jnp.float32),
    )(params, idx_flat, cnt_flat)
    return jnp.transpose(img, (1, 2, 0))
